# baseline (speedup 1.0000x reference)
import functools

import jax
import jax.numpy as jnp
from jax import lax
from jax.experimental import pallas as pl
from jax.experimental.pallas import tpu as pltpu
from jax.experimental.pallas import tpu_sc as plsc

N_CELLS = 4194304
NUM_TYPES = 19
N_SC_TBL = 3
NC, NS, L = 2, 16, 16
NW = NC * NS
PER_W = N_CELLS // NW
CHUNK = 8192
NCHUNK = PER_W // CHUNK
NGRP = NCHUNK // 2
TBL_PAD = 64

TC_BLK = 524288


def _sc_body(idx_hbm, tbl_hbm, o0, o1, o2,
             tbl_v, ib0, ib1,
             ob00, ob01, ob02,
             ob10, ob11, ob12,
             sin0, sin1, sout0, sout1):
    wid = lax.axis_index("s") * NC + lax.axis_index("c")
    base_w = wid * PER_W
    pltpu.sync_copy(tbl_hbm, tbl_v)
    outs = (o0, o1, o2)
    ibufs = (ib0, ib1)
    obufs = ((ob00, ob01, ob02), (ob10, ob11, ob12))
    sins = (sin0, sin1)
    souts = (sout0, sout1)

    for b in range(2):
        pltpu.async_copy(
            idx_hbm.at[pl.ds(base_w + b * CHUNK, CHUNK)], ibufs[b], sins[b]
        )

    def grp_body(g, carry):
        for b in range(2):
            ci = 2 * g + b
            base = pl.multiple_of(base_w + ci * CHUNK, CHUNK)
            pltpu.make_async_copy(
                idx_hbm.at[pl.ds(base, CHUNK)], ibufs[b], sins[b]
            ).wait()

            @pl.when(g > 0)
            def _drain():
                prev = pl.multiple_of(base - 2 * CHUNK, CHUNK)
                for t in range(N_SC_TBL):
                    pltpu.make_async_copy(
                        obufs[b][t], outs[t].at[pl.ds(prev, CHUNK)], souts[b]
                    ).wait()

            @plsc.parallel_loop(0, CHUNK, step=L, unroll=8)
            def _gather(off):
                iv = ibufs[b][pl.ds(off, L)]
                for t in range(N_SC_TBL):
                    obufs[b][t][pl.ds(off, L)] = plsc.load_gather(
                        tbl_v, [iv + (t * NUM_TYPES - 1)]
                    )

            @pl.when(ci + 2 < NCHUNK)
            def _prefetch():
                nbase = pl.multiple_of(base + 2 * CHUNK, CHUNK)
                pltpu.async_copy(
                    idx_hbm.at[pl.ds(nbase, CHUNK)], ibufs[b], sins[b]
                )

            for t in range(N_SC_TBL):
                pltpu.async_copy(
                    obufs[b][t], outs[t].at[pl.ds(base, CHUNK)], souts[b]
                )
        return carry

    lax.fori_loop(0, NGRP, grp_body, 0)

    for b in range(2):
        ci = NCHUNK - 2 + b
        base = pl.multiple_of(base_w + ci * CHUNK, CHUNK)
        for t in range(N_SC_TBL):
            pltpu.make_async_copy(
                obufs[b][t], outs[t].at[pl.ds(base, CHUNK)], souts[b]
            ).wait()


def _tc_body(*refs):
    n = (len(refs) - 1) // 2
    t_refs, idx_ref, o_refs = refs[:n], refs[n], refs[n + 1:]
    rows = TC_BLK // 128
    idx = idx_ref[...].reshape(rows, 128) - 1
    for t_ref, o_ref in zip(t_refs, o_refs):
        row = t_ref[...]
        x = jnp.broadcast_to(row[None, :], (rows, 128))
        g = jnp.take_along_axis(x, idx, axis=1, mode="promise_in_bounds")
        o_ref[...] = g.reshape(TC_BLK)


def _tc_lookup(tables, idx):
    n = len(tables)
    grid = (N_CELLS // TC_BLK,)
    blk = pl.BlockSpec((TC_BLK,), lambda i: (i,))
    return pl.pallas_call(
        _tc_body,
        grid=grid,
        in_specs=[pl.BlockSpec(memory_space=pltpu.VMEM)] * n + [blk],
        out_specs=[blk] * n,
        out_shape=[jax.ShapeDtypeStruct((N_CELLS,), jnp.float32)] * n,
        compiler_params=pltpu.CompilerParams(
            dimension_semantics=("arbitrary",)
        ),
    )(*tables, idx)


@jax.jit
def kernel(indices, BB, MAXSMC, SATDK, SATPSI, QTZ):
    tbl = jnp.concatenate(
        [BB, MAXSMC, SATDK,
         jnp.zeros((TBL_PAD - N_SC_TBL * NUM_TYPES,), jnp.float32)]
    )
    mesh = plsc.VectorSubcoreMesh(
        core_axis_name="c", subcore_axis_name="s", num_cores=NC, num_subcores=NS
    )
    out = jax.ShapeDtypeStruct((N_CELLS,), jnp.float32)
    sc = pl.kernel(
        _sc_body,
        out_type=(out,) * N_SC_TBL,
        mesh=mesh,
        scratch_types=(
            [pltpu.VMEM((TBL_PAD,), jnp.float32)]
            + [pltpu.VMEM((CHUNK,), jnp.int32)] * 2
            + [pltpu.VMEM((CHUNK,), jnp.float32)] * (2 * N_SC_TBL)
            + [pltpu.SemaphoreType.DMA] * 4
        ),
        compiler_params=pltpu.CompilerParams(needs_layout_passes=False),
    )
    del sc, tbl, mesh
    pad = jnp.zeros((128 - NUM_TYPES,), jnp.float32)
    bb, maxsmc, satdk, satpsi, qtz = _tc_lookup(
        [jnp.concatenate([T, pad]) for T in (BB, MAXSMC, SATDK, SATPSI, QTZ)],
        indices,
    )
    return (bb, maxsmc, satdk, satpsi, qtz)


# speedup vs baseline: 1.1552x; 1.1552x over previous
import functools

import jax
import jax.numpy as jnp
from jax import lax
from jax.experimental import pallas as pl
from jax.experimental.pallas import tpu as pltpu
from jax.experimental.pallas import tpu_sc as plsc

N_CELLS = 4194304
NUM_TYPES = 19
NC, NS, L = 2, 16, 16
NW = NC * NS
PER_W = N_CELLS // NW
CHUNK = 8192
NCHUNK = PER_W // CHUNK
NGRP = NCHUNK // 2
TBL_PAD = 96


def _sc_body(idx_hbm, tbl_hbm, o0, o1, o2, o3, o4,
             tbl_v, ib0, ib1,
             ob00, ob01, ob02, ob03, ob04,
             ob10, ob11, ob12, ob13, ob14,
             sin0, sin1, sout0, sout1):
    wid = lax.axis_index("s") * NC + lax.axis_index("c")
    base_w = wid * PER_W
    pltpu.sync_copy(tbl_hbm, tbl_v)
    outs = (o0, o1, o2, o3, o4)
    ibufs = (ib0, ib1)
    obufs = ((ob00, ob01, ob02, ob03, ob04), (ob10, ob11, ob12, ob13, ob14))
    sins = (sin0, sin1)
    souts = (sout0, sout1)

    for b in range(2):
        pltpu.async_copy(
            idx_hbm.at[pl.ds(base_w + b * CHUNK, CHUNK)], ibufs[b], sins[b]
        )

    def grp_body(g, carry):
        for b in range(2):
            ci = 2 * g + b
            base = pl.multiple_of(base_w + ci * CHUNK, CHUNK)
            pltpu.make_async_copy(
                idx_hbm.at[pl.ds(base, CHUNK)], ibufs[b], sins[b]
            ).wait()

            @pl.when(g > 0)
            def _drain():
                prev = pl.multiple_of(base - 2 * CHUNK, CHUNK)
                for t in range(5):
                    pltpu.make_async_copy(
                        obufs[b][t], outs[t].at[pl.ds(prev, CHUNK)], souts[b]
                    ).wait()

            @plsc.parallel_loop(0, CHUNK, step=L, unroll=8)
            def _gather(off):
                iv = ibufs[b][pl.ds(off, L)]
                for t in range(5):
                    obufs[b][t][pl.ds(off, L)] = plsc.load_gather(
                        tbl_v, [iv + (t * NUM_TYPES - 1)]
                    )

            @pl.when(ci + 2 < NCHUNK)
            def _prefetch():
                nbase = pl.multiple_of(base + 2 * CHUNK, CHUNK)
                pltpu.async_copy(
                    idx_hbm.at[pl.ds(nbase, CHUNK)], ibufs[b], sins[b]
                )

            for t in range(5):
                pltpu.async_copy(
                    obufs[b][t], outs[t].at[pl.ds(base, CHUNK)], souts[b]
                )
        return carry

    lax.fori_loop(0, NGRP, grp_body, 0)

    for b in range(2):
        ci = NCHUNK - 2 + b
        base = pl.multiple_of(base_w + ci * CHUNK, CHUNK)
        for t in range(5):
            pltpu.make_async_copy(
                obufs[b][t], outs[t].at[pl.ds(base, CHUNK)], souts[b]
            ).wait()


@jax.jit
def kernel(indices, BB, MAXSMC, SATDK, SATPSI, QTZ):
    tbl = jnp.concatenate(
        [BB, MAXSMC, SATDK, SATPSI, QTZ,
         jnp.zeros((TBL_PAD - 5 * NUM_TYPES,), jnp.float32)]
    )
    mesh = plsc.VectorSubcoreMesh(
        core_axis_name="c", subcore_axis_name="s", num_cores=NC, num_subcores=NS
    )
    out = jax.ShapeDtypeStruct((N_CELLS,), jnp.float32)
    f = pl.kernel(
        _sc_body,
        out_type=(out,) * 5,
        mesh=mesh,
        scratch_types=(
            [pltpu.VMEM((TBL_PAD,), jnp.float32)]
            + [pltpu.VMEM((CHUNK,), jnp.int32)] * 2
            + [pltpu.VMEM((CHUNK,), jnp.float32)] * 10
            + [pltpu.SemaphoreType.DMA] * 4
        ),
        compiler_params=pltpu.CompilerParams(needs_layout_passes=False),
    )
    return f(indices, tbl)
